# Initial kernel scaffold; baseline (speedup 1.0000x reference)
import jax
import jax.numpy as jnp
from jax.experimental import pallas as pl


def kernel(score, target):
    raise NotImplementedError("write your pallas kernel here")


# speedup vs baseline: 5.5232x; 5.5232x over previous
import functools

import jax
import jax.numpy as jnp
from jax import lax
from jax.experimental import pallas as pl

_NUM_CLASSES = 5


def _ceil_pow2(n):
    p = 1
    while p < n:
        p *= 2
    return p


def _lovasz_body(score_ref, target_ref, out_ref, *, rows_in, rows_pad, npix):
    cls = pl.program_id(0)
    row_bits = rows_pad.bit_length() - 1
    n_total = rows_pad * 128

    s = score_ref[0, 0]
    t = target_ref[0]
    lab_i = jnp.where(t == cls, 1, 0).astype(jnp.int32)
    lab_f = lab_i.astype(jnp.float32)
    sign = 2.0 * lab_f - 1.0
    err = 1.0 - s * sign

    u = lax.bitcast_convert_type(err, jnp.int32)
    u = (u & jnp.int32(-2)) | lab_i
    x = jnp.where(u < 0, u ^ jnp.int32(0x7FFFFFFF), u)
    keys = ~x

    pad = jnp.full((rows_pad - rows_in, 128), jnp.int32(0x7FFFFFFF))
    k = jnp.concatenate([keys, pad], axis=0)

    row_iota = lax.broadcasted_iota(jnp.int32, (rows_pad, 128), 0)
    col_iota = lax.broadcasted_iota(jnp.int32, (rows_pad, 128), 1)

    def compare_exchange(v, k2, j):
        if j < rows_pad:
            axis, amt, bit_src = 0, j, row_iota & j
        else:
            axis, amt, bit_src = 1, j >> row_bits, col_iota & (j >> row_bits)
        low = bit_src == 0
        partner = jnp.where(low,
                            jnp.roll(v, -amt, axis=axis),
                            jnp.roll(v, amt, axis=axis))
        if k2 >= n_total:
            up = jnp.full((rows_pad, 128), True)
        elif k2 < rows_pad:
            up = (row_iota & k2) == 0
        else:
            up = (col_iota & (k2 >> row_bits)) == 0
        take_min = low == up
        return jnp.where(take_min, jnp.minimum(v, partner),
                         jnp.maximum(v, partner))

    k2 = 2
    while k2 <= n_total:
        j = k2 // 2
        while j >= 1:
            k = compare_exchange(k, k2, j)
            j //= 2
        k2 *= 2

    xu = ~k
    u2 = jnp.where(xu < 0, xu ^ jnp.int32(0x7FFFFFFF), xu)
    g = (u2 & 1).astype(jnp.float32)
    e = lax.bitcast_convert_type(u2, jnp.float32)

    lin = col_iota * rows_pad + row_iota
    valid = lin < npix
    g = jnp.where(valid, g, 0.0)
    m = jnp.sum(g)

    cs = g
    sh = 1
    while sh < rows_pad:
        cs = cs + jnp.where(row_iota >= sh, jnp.roll(cs, sh, axis=0), 0.0)
        sh *= 2
    coltot = lax.slice(cs, (rows_pad - 1, 0), (rows_pad, 128))
    cc = coltot
    sh = 1
    while sh < 128:
        cc = cc + jnp.where(lax.broadcasted_iota(jnp.int32, (1, 128), 1) >= sh,
                            jnp.roll(cc, sh, axis=1), 0.0)
        sh *= 2
    col_excl = cc - coltot
    cg = cs + col_excl

    tpos = (lin + 1).astype(jnp.float32)
    npix_f = jnp.float32(npix)
    m2 = npix_f - m
    cg2 = tpos - cg
    jac1 = 1.0 - (m - cg) / (m + tpos - cg)
    jac2 = 1.0 - (m2 - cg2) / (m2 + tpos - cg2)
    js = jac1 + jac2

    rolled = jnp.roll(js, 1, axis=0)
    last = lax.slice(js, (rows_pad - 1, 0), (rows_pad, 128))
    last_sh = jnp.roll(last, 1, axis=1)
    last_sh = jnp.where(lax.broadcasted_iota(jnp.int32, (1, 128), 1) == 0,
                        0.0, last_sh)
    prev = jnp.where(row_iota == 0, last_sh, rolled)

    f = jnp.where(e > 0, e + 1.0, jnp.exp(jnp.minimum(e, 0.0)))
    contrib = jnp.where(valid, f * (js - prev), 0.0)
    out_ref[0, 0] = (0.5 * jnp.sum(contrib)).reshape(1, 1)


def kernel(score, target):
    b, c, h, w = score.shape
    npix = h * w
    rows_in = npix // 128
    rows_pad = _ceil_pow2(rows_in)
    score2 = score.reshape(b, c, rows_in, 128)
    target2 = target.astype(jnp.int32).reshape(b, rows_in, 128)

    body = functools.partial(_lovasz_body, rows_in=rows_in,
                             rows_pad=rows_pad, npix=npix)
    out = pl.pallas_call(
        body,
        grid=(c, b),
        in_specs=[
            pl.BlockSpec((1, 1, rows_in, 128), lambda ci, bi: (bi, ci, 0, 0)),
            pl.BlockSpec((1, rows_in, 128), lambda ci, bi: (bi, 0, 0)),
        ],
        out_specs=pl.BlockSpec((1, 1, 1, 1), lambda ci, bi: (ci, bi, 0, 0)),
        out_shape=jax.ShapeDtypeStruct((c, b, 1, 1), jnp.float32),
    )(score2, target2)
    return tuple(out[i].reshape(b, 1) for i in range(c))


# speedup vs baseline: 5.7209x; 1.0358x over previous
import functools

import jax
import jax.numpy as jnp
from jax import lax
from jax.experimental import pallas as pl

_NUM_CLASSES = 5


def _ceil_pow2(n):
    p = 1
    while p < n:
        p *= 2
    return p


def _lovasz_body(score_ref, target_ref, out_ref, *, rows_in, rows_pad, npix):
    cls = pl.program_id(0)
    row_bits = rows_pad.bit_length() - 1
    n_total = rows_pad * 128

    s = score_ref[0, 0]
    t = target_ref[0]
    lab_i = jnp.where(t == cls, 1, 0).astype(jnp.int32)
    lab_f = lab_i.astype(jnp.float32)
    sign = 2.0 * lab_f - 1.0
    err = 1.0 - s * sign

    u = lax.bitcast_convert_type(err, jnp.int32)
    u = (u & jnp.int32(-2)) | lab_i
    x = jnp.where(u < 0, u ^ jnp.int32(0x7FFFFFFF), u)
    keys = ~x

    pad = jnp.full((rows_pad - rows_in, 128), jnp.int32(0x7FFFFFFF))
    k = jnp.concatenate([keys, pad], axis=0)

    row_iota = lax.broadcasted_iota(jnp.int32, (rows_pad, 128), 0)
    col_iota = lax.broadcasted_iota(jnp.int32, (rows_pad, 128), 1)

    def compare_exchange(v, k2, j):
        if 8 <= j < rows_pad:
            grp = rows_pad // (2 * j)
            v4 = v.reshape(grp, 2, j, 128)
            lo = v4[:, 0]
            hi = v4[:, 1]
            mn = jnp.minimum(lo, hi)
            mx = jnp.maximum(lo, hi)
            if k2 >= n_total:
                up = jnp.full((1, 1, 1), True)
            elif k2 < rows_pad:
                up = (lax.broadcasted_iota(jnp.int32, (grp, 1, 1), 0)
                      & (k2 // (2 * j))) == 0
            else:
                up = (lax.broadcasted_iota(jnp.int32, (1, 1, 128), 2)
                      & (k2 >> row_bits)) == 0
            new_lo = jnp.where(up, mn, mx)
            new_hi = jnp.where(up, mx, mn)
            return jnp.concatenate([new_lo[:, None], new_hi[:, None]],
                                   axis=1).reshape(rows_pad, 128)
        if j < rows_pad:
            axis, amt, bit_src = 0, j, row_iota & j
        else:
            axis, amt, bit_src = 1, j >> row_bits, col_iota & (j >> row_bits)
        low = bit_src == 0
        partner = jnp.where(low,
                            jnp.roll(v, -amt, axis=axis),
                            jnp.roll(v, amt, axis=axis))
        if k2 >= n_total:
            up = jnp.full((rows_pad, 128), True)
        elif k2 < rows_pad:
            up = (row_iota & k2) == 0
        else:
            up = (col_iota & (k2 >> row_bits)) == 0
        take_min = low == up
        return jnp.where(take_min, jnp.minimum(v, partner),
                         jnp.maximum(v, partner))

    k2 = 2
    while k2 <= n_total:
        j = k2 // 2
        while j >= 1:
            k = compare_exchange(k, k2, j)
            j //= 2
        k2 *= 2

    xu = ~k
    u2 = jnp.where(xu < 0, xu ^ jnp.int32(0x7FFFFFFF), xu)
    g = (u2 & 1).astype(jnp.float32)
    e = lax.bitcast_convert_type(u2, jnp.float32)

    lin = col_iota * rows_pad + row_iota
    valid = lin < npix
    g = jnp.where(valid, g, 0.0)
    m = jnp.sum(g)

    cs = g
    sh = 1
    while sh < rows_pad:
        cs = cs + jnp.where(row_iota >= sh, jnp.roll(cs, sh, axis=0), 0.0)
        sh *= 2
    coltot = lax.slice(cs, (rows_pad - 1, 0), (rows_pad, 128))
    cc = coltot
    sh = 1
    while sh < 128:
        cc = cc + jnp.where(lax.broadcasted_iota(jnp.int32, (1, 128), 1) >= sh,
                            jnp.roll(cc, sh, axis=1), 0.0)
        sh *= 2
    col_excl = cc - coltot
    cg = cs + col_excl

    tpos = (lin + 1).astype(jnp.float32)
    npix_f = jnp.float32(npix)
    m2 = npix_f - m
    cg2 = tpos - cg
    jac1 = 1.0 - (m - cg) / (m + tpos - cg)
    jac2 = 1.0 - (m2 - cg2) / (m2 + tpos - cg2)
    js = jac1 + jac2

    rolled = jnp.roll(js, 1, axis=0)
    last = lax.slice(js, (rows_pad - 1, 0), (rows_pad, 128))
    last_sh = jnp.roll(last, 1, axis=1)
    last_sh = jnp.where(lax.broadcasted_iota(jnp.int32, (1, 128), 1) == 0,
                        0.0, last_sh)
    prev = jnp.where(row_iota == 0, last_sh, rolled)

    f = jnp.where(e > 0, e + 1.0, jnp.exp(jnp.minimum(e, 0.0)))
    contrib = jnp.where(valid, f * (js - prev), 0.0)
    out_ref[0, 0] = (0.5 * jnp.sum(contrib)).reshape(1, 1)


def kernel(score, target):
    b, c, h, w = score.shape
    npix = h * w
    rows_in = npix // 128
    rows_pad = _ceil_pow2(rows_in)
    score2 = score.reshape(b, c, rows_in, 128)
    target2 = target.astype(jnp.int32).reshape(b, rows_in, 128)

    body = functools.partial(_lovasz_body, rows_in=rows_in,
                             rows_pad=rows_pad, npix=npix)
    out = pl.pallas_call(
        body,
        grid=(c, b),
        in_specs=[
            pl.BlockSpec((1, 1, rows_in, 128), lambda ci, bi: (bi, ci, 0, 0)),
            pl.BlockSpec((1, rows_in, 128), lambda ci, bi: (bi, 0, 0)),
        ],
        out_specs=pl.BlockSpec((1, 1, 1, 1), lambda ci, bi: (ci, bi, 0, 0)),
        out_shape=jax.ShapeDtypeStruct((c, b, 1, 1), jnp.float32),
    )(score2, target2)
    return tuple(out[i].reshape(b, 1) for i in range(c))


# speedup vs baseline: 8.7720x; 1.5333x over previous
import functools

import jax
import jax.numpy as jnp
import numpy as np
from jax import lax
from jax.experimental import pallas as pl

_MAXK = np.int32(0x7FFFFFFF)


def _keys_from(s, t, cls):
    lab_i = jnp.where(t == cls, 1, 0).astype(jnp.int32)
    sign = 2.0 * lab_i.astype(jnp.float32) - 1.0
    err = 1.0 - s * sign
    u = lax.bitcast_convert_type(err, jnp.int32)
    u = (u & jnp.int32(-2)) | lab_i
    x = jnp.where(u < 0, u ^ _MAXK, u)
    return ~x


def _ce(v, k2, j, nrows, row_bits, seg_bits, desc=False):
    row_iota = lax.broadcasted_iota(jnp.int32, (nrows, 128), 0)
    col_iota = lax.broadcasted_iota(jnp.int32, (nrows, 128), 1)
    seg = 1 << seg_bits

    def updir(shape, riota, ciota):
        if k2 == 0:
            return jnp.full(shape, True)
        if k2 < seg:
            return (riota & k2) == 0
        return (ciota & (k2 >> seg_bits)) == 0

    if 8 <= j < seg:
        grp = nrows // (2 * j)
        v4 = v.reshape(grp, 2, j, 128)
        lo = v4[:, 0]
        hi = v4[:, 1]
        mn = jnp.minimum(lo, hi)
        mx = jnp.maximum(lo, hi)
        if k2 == 0:
            up = jnp.full((1, 1, 1), True)
        elif k2 < seg:
            up = (lax.broadcasted_iota(jnp.int32, (grp, 1, 1), 0)
                  & (k2 // (2 * j))) == 0
        else:
            up = (lax.broadcasted_iota(jnp.int32, (1, 1, 128), 2)
                  & (k2 >> seg_bits)) == 0
        if desc:
            up = ~up
        new_lo = jnp.where(up, mn, mx)
        new_hi = jnp.where(up, mx, mn)
        return jnp.concatenate([new_lo[:, None], new_hi[:, None]],
                               axis=1).reshape(nrows, 128)

    if j < seg:
        axis, amt, bit_src = 0, j, row_iota & j
    else:
        axis, amt, bit_src = 1, j >> seg_bits, col_iota & (j >> seg_bits)
    low = bit_src == 0
    partner = jnp.where(low,
                        jnp.roll(v, -amt, axis=axis),
                        jnp.roll(v, amt, axis=axis))
    up = updir((nrows, 128), row_iota, col_iota)
    if desc:
        up = ~up
    take_min = low == up
    return jnp.where(take_min, jnp.minimum(v, partner),
                     jnp.maximum(v, partner))


def _bitonic(v, n_bits, nrows, seg_bits, desc=False):
    for s in range(1, n_bits + 1):
        k2 = 0 if s == n_bits else (1 << s)
        j = 1 << (s - 1)
        while j >= 1:
            v = _ce(v, k2, j, nrows, seg_bits=seg_bits, row_bits=None,
                    desc=desc)
            j //= 2
    return v


def _sort_a_body(score_ref, t0_ref, t1_ref, out_ref):
    q = pl.program_id(0)
    keys = []
    for p, t_ref in ((0, t0_ref), (1, t1_ref)):
        s = score_ref[p][:1024]
        t = t_ref[0][:1024]
        cls = (2 * q + p) % 5
        keys.append(_keys_from(s, t, cls))
    v = jnp.concatenate(keys, axis=0)
    v = _bitonic(v, 17, 2048, seg_bits=10)
    out_ref[...] = v.reshape(1, 2048, 128)


def _sort_b_body(score_ref, target_ref, out_ref):
    g = pl.program_id(0)
    cols = []
    for p in range(8):
        q = 8 * g + p
        s = score_ref[p]
        t = target_ref[q // 5]
        k = _keys_from(s, t, q % 5)
        strips = [lax.slice(k, (0, 16 * j), (128, 16 * j + 16))
                  for j in range(8)]
        cols.append(jnp.concatenate(strips, axis=0))
    v = jnp.concatenate(cols, axis=1)
    v = _bitonic(v, 14, 1024, seg_bits=10, desc=True)
    for p in range(8):
        out_ref[0, p] = lax.slice(v, (0, 16 * p), (1024, 16 * p + 16))


def _merge_body(a_ref, b_ref, out_ref, *, npix):
    a = a_ref[0]
    btail = b_ref[0, 0]
    pad = jnp.full((1024, 112), _MAXK)
    tail = jnp.concatenate([pad, btail], axis=1)
    v = jnp.concatenate([a, tail], axis=0)

    row_iota = lax.broadcasted_iota(jnp.int32, (2048, 128), 0)
    col_iota = lax.broadcasted_iota(jnp.int32, (2048, 128), 1)
    top = v[:1024]
    bot = v[1024:]
    v = jnp.concatenate([jnp.minimum(top, bot), jnp.maximum(top, bot)],
                        axis=0)
    for m in (64, 32, 16, 8, 4, 2, 1):
        low = (col_iota & m) == 0
        partner = jnp.where(low, jnp.roll(v, -m, axis=1),
                            jnp.roll(v, m, axis=1))
        v = jnp.where(low, jnp.minimum(v, partner),
                      jnp.maximum(v, partner))
    j = 512
    while j >= 1:
        v = _ce(v, 0, j, 2048, seg_bits=10, row_bits=None)
        j //= 2

    xu = ~v
    u2 = jnp.where(xu < 0, xu ^ _MAXK, xu)
    g = (u2 & 1).astype(jnp.float32)
    e = lax.bitcast_convert_type(u2, jnp.float32)

    r10 = row_iota & 1023
    half = row_iota >> 10
    t = r10 + (col_iota << 10) + (half << 17)
    valid = t < npix
    g = jnp.where(valid, g, 0.0)
    m_tot = jnp.sum(g)

    cs = g
    sh = 1
    while sh < 1024:
        cs = cs + jnp.where(r10 >= sh, jnp.roll(cs, sh, axis=0), 0.0)
        sh *= 2
    seg_tot = jnp.concatenate([lax.slice(cs, (1023, 0), (1024, 128)),
                               lax.slice(cs, (2047, 0), (2048, 128))],
                              axis=0)
    lane2 = lax.broadcasted_iota(jnp.int32, (2, 128), 1)
    half2 = lax.broadcasted_iota(jnp.int32, (2, 128), 0)
    inc = seg_tot
    sh = 1
    while sh < 128:
        inc = inc + jnp.where(lane2 >= sh, jnp.roll(inc, sh, axis=1), 0.0)
        sh *= 2
    row0_tot = lax.slice(inc, (0, 127), (1, 128))
    inc = inc + jnp.where(half2 == 1, row0_tot, 0.0)
    seg_excl = inc - seg_tot
    cg = cs + jnp.repeat(seg_excl.reshape(2, 1, 128), 1024,
                         axis=1).reshape(2048, 128)

    tpos = (t + 1).astype(jnp.float32)
    npix_f = jnp.float32(npix)
    m2 = npix_f - m_tot
    cg2 = tpos - cg
    jac1 = 1.0 - (m_tot - cg) / (m_tot + tpos - cg)
    jac2 = 1.0 - (m2 - cg2) / (m2 + tpos - cg2)
    js = jac1 + jac2

    last_vals = jnp.concatenate([lax.slice(js, (1023, 0), (1024, 128)),
                                 lax.slice(js, (2047, 0), (2048, 128))],
                                axis=0)
    rl = jnp.roll(last_vals, 1, axis=1)
    lv_h0_last = lax.slice(last_vals, (0, 127), (1, 128))
    prev_last = jnp.where(lane2 == 0,
                          jnp.where(half2 == 0, 0.0, lv_h0_last), rl)
    prev = jnp.where(r10 == 0,
                     jnp.repeat(prev_last.reshape(2, 1, 128), 1024,
                                axis=1).reshape(2048, 128),
                     jnp.roll(js, 1, axis=0))

    f = jnp.where(e > 0, e + 1.0, jnp.exp(jnp.minimum(e, 0.0)))
    contrib = jnp.where(valid, f * (js - prev), 0.0)
    out_ref[0, 0] = (0.5 * jnp.sum(contrib)).reshape(1, 1)


def kernel(score, target):
    b, c, h, w = score.shape
    npix = h * w
    nq = b * c
    score3 = score.reshape(nq, npix // 128, 128)
    target3 = target.astype(jnp.int32).reshape(b, npix // 128, 128)

    a_sorted = pl.pallas_call(
        _sort_a_body,
        grid=(nq // 2,),
        in_specs=[
            pl.BlockSpec((2, 1152, 128), lambda q: (q, 0, 0)),
            pl.BlockSpec((1, 1152, 128), lambda q: ((2 * q) // 5, 0, 0)),
            pl.BlockSpec((1, 1152, 128), lambda q: ((2 * q + 1) // 5, 0, 0)),
        ],
        out_specs=pl.BlockSpec((1, 2048, 128), lambda q: (q, 0, 0)),
        out_shape=jax.ShapeDtypeStruct((nq // 2, 2048, 128), jnp.int32),
    )(score3, target3, target3)
    a_sorted = a_sorted.reshape(nq, 1024, 128)

    b_sorted = pl.pallas_call(
        _sort_b_body,
        grid=(nq // 8,),
        in_specs=[
            pl.BlockSpec((8, 128, 128), lambda g: (g, 8, 0)),
            pl.BlockSpec((8, 128, 128), lambda g: (0, 8, 0)),
        ],
        out_specs=pl.BlockSpec((1, 8, 1024, 16), lambda g: (g, 0, 0, 0)),
        out_shape=jax.ShapeDtypeStruct((nq // 8, 8, 1024, 16), jnp.int32),
    )(score3, target3)

    out = pl.pallas_call(
        functools.partial(_merge_body, npix=npix),
        grid=(nq,),
        in_specs=[
            pl.BlockSpec((1, 1024, 128), lambda q: (q, 0, 0)),
            pl.BlockSpec((1, 1, 1024, 16), lambda q: (q // 8, q % 8, 0, 0)),
        ],
        out_specs=pl.BlockSpec((1, 1, 1, 1), lambda q: (q % 5, q // 5, 0, 0)),
        out_shape=jax.ShapeDtypeStruct((c, b, 1, 1), jnp.float32),
    )(a_sorted, b_sorted)
    return tuple(out[i].reshape(b, 1) for i in range(c))


# speedup vs baseline: 8.9037x; 1.0150x over previous
import functools

import jax
import jax.numpy as jnp
import numpy as np
from jax import lax
from jax.experimental import pallas as pl

_MAXK = np.int32(0x7FFFFFFF)


def _keys_from(s, t, cls):
    lab_i = jnp.where(t == cls, 1, 0).astype(jnp.int32)
    sign = 2.0 * lab_i.astype(jnp.float32) - 1.0
    err = 1.0 - s * sign
    u = lax.bitcast_convert_type(err, jnp.int32)
    u = (u & jnp.int32(-2)) | lab_i
    x = jnp.where(u < 0, u ^ _MAXK, u)
    return ~x


def _ce(v, k2, j, nrows, row_bits, seg_bits, desc=False):
    row_iota = lax.broadcasted_iota(jnp.int32, (nrows, 128), 0)
    col_iota = lax.broadcasted_iota(jnp.int32, (nrows, 128), 1)
    seg = 1 << seg_bits

    def updir(shape, riota, ciota):
        if k2 == 0:
            return jnp.full(shape, True)
        if k2 < seg:
            return (riota & k2) == 0
        return (ciota & (k2 >> seg_bits)) == 0

    if 8 <= j < seg:
        grp = nrows // (2 * j)
        v4 = v.reshape(grp, 2, j, 128)
        lo = v4[:, 0]
        hi = v4[:, 1]
        mn = jnp.minimum(lo, hi)
        mx = jnp.maximum(lo, hi)
        if k2 == 0:
            up = jnp.full((1, 1, 1), True)
        elif k2 < seg:
            up = (lax.broadcasted_iota(jnp.int32, (grp, 1, 1), 0)
                  & (k2 // (2 * j))) == 0
        else:
            up = (lax.broadcasted_iota(jnp.int32, (1, 1, 128), 2)
                  & (k2 >> seg_bits)) == 0
        if desc:
            up = ~up
        new_lo = jnp.where(up, mn, mx)
        new_hi = jnp.where(up, mx, mn)
        return jnp.concatenate([new_lo[:, None], new_hi[:, None]],
                               axis=1).reshape(nrows, 128)

    if j < seg:
        axis, amt, bit_src = 0, j, row_iota & j
    else:
        axis, amt, bit_src = 1, j >> seg_bits, col_iota & (j >> seg_bits)
    low = bit_src == 0
    rm = jnp.roll(v, -amt, axis=axis)
    rp = jnp.roll(v, amt, axis=axis)
    if k2 == 0:
        if desc:
            return jnp.where(low, jnp.maximum(v, rm), jnp.minimum(v, rp))
        return jnp.where(low, jnp.minimum(v, rm), jnp.maximum(v, rp))
    partner = jnp.where(low, rm, rp)
    up = updir((nrows, 128), row_iota, col_iota)
    if desc:
        up = ~up
    take_min = low == up
    return jnp.where(take_min, jnp.minimum(v, partner),
                     jnp.maximum(v, partner))


def _bitonic(v, n_bits, nrows, seg_bits, desc=False):
    for s in range(1, n_bits + 1):
        k2 = 0 if s == n_bits else (1 << s)
        j = 1 << (s - 1)
        while j >= 1:
            v = _ce(v, k2, j, nrows, seg_bits=seg_bits, row_bits=None,
                    desc=desc)
            j //= 2
    return v


def _sort_a_body(score_ref, t0_ref, t1_ref, out_ref):
    q = pl.program_id(0)
    keys = []
    for p, t_ref in ((0, t0_ref), (1, t1_ref)):
        s = score_ref[p][:1024]
        t = t_ref[0][:1024]
        cls = (2 * q + p) % 5
        keys.append(_keys_from(s, t, cls))
    v = jnp.concatenate(keys, axis=0)
    v = _bitonic(v, 17, 2048, seg_bits=10)
    out_ref[...] = v.reshape(1, 2048, 128)


def _sort_b_body(score_ref, target_ref, out_ref):
    g = pl.program_id(0)
    cols = []
    for p in range(8):
        q = 8 * g + p
        s = score_ref[p]
        t = target_ref[q // 5]
        k = _keys_from(s, t, q % 5)
        strips = [lax.slice(k, (0, 16 * j), (128, 16 * j + 16))
                  for j in range(8)]
        cols.append(jnp.concatenate(strips, axis=0))
    v = jnp.concatenate(cols, axis=1)
    v = _bitonic(v, 14, 1024, seg_bits=10, desc=True)
    for p in range(8):
        out_ref[0, p] = lax.slice(v, (0, 16 * p), (1024, 16 * p + 16))


def _merge_body(a_ref, b_ref, out_ref, *, npix):
    a = a_ref[0]
    btail = b_ref[0, 0]
    pad = jnp.full((1024, 112), _MAXK)
    tail = jnp.concatenate([pad, btail], axis=1)
    v = jnp.concatenate([a, tail], axis=0)

    row_iota = lax.broadcasted_iota(jnp.int32, (2048, 128), 0)
    col_iota = lax.broadcasted_iota(jnp.int32, (2048, 128), 1)
    top = v[:1024]
    bot = v[1024:]
    v = jnp.concatenate([jnp.minimum(top, bot), jnp.maximum(top, bot)],
                        axis=0)
    for m in (64, 32, 16, 8, 4, 2, 1):
        low = (col_iota & m) == 0
        v = jnp.where(low, jnp.minimum(v, jnp.roll(v, -m, axis=1)),
                      jnp.maximum(v, jnp.roll(v, m, axis=1)))
    j = 512
    while j >= 1:
        v = _ce(v, 0, j, 2048, seg_bits=10, row_bits=None)
        j //= 2

    xu = ~v
    u2 = jnp.where(xu < 0, xu ^ _MAXK, xu)
    g = (u2 & 1).astype(jnp.float32)
    e = lax.bitcast_convert_type(u2, jnp.float32)

    r10 = row_iota & 1023
    half = row_iota >> 10
    t = r10 + (col_iota << 10) + (half << 17)
    valid = t < npix
    g = jnp.where(valid, g, 0.0)
    m_tot = jnp.sum(g)

    cs = g
    sh = 1
    while sh < 1024:
        cs = cs + jnp.where(r10 >= sh, jnp.roll(cs, sh, axis=0), 0.0)
        sh *= 2
    seg_tot = jnp.concatenate([lax.slice(cs, (1023, 0), (1024, 128)),
                               lax.slice(cs, (2047, 0), (2048, 128))],
                              axis=0)
    lane2 = lax.broadcasted_iota(jnp.int32, (2, 128), 1)
    half2 = lax.broadcasted_iota(jnp.int32, (2, 128), 0)
    inc = seg_tot
    sh = 1
    while sh < 128:
        inc = inc + jnp.where(lane2 >= sh, jnp.roll(inc, sh, axis=1), 0.0)
        sh *= 2
    row0_tot = lax.slice(inc, (0, 127), (1, 128))
    inc = inc + jnp.where(half2 == 1, row0_tot, 0.0)
    seg_excl = inc - seg_tot
    cg = cs + jnp.repeat(seg_excl.reshape(2, 1, 128), 1024,
                         axis=1).reshape(2048, 128)

    tpos = (t + 1).astype(jnp.float32)
    npix_f = jnp.float32(npix)
    m2 = npix_f - m_tot
    cg2 = tpos - cg
    jac1 = 1.0 - (m_tot - cg) / (m_tot + tpos - cg)
    jac2 = 1.0 - (m2 - cg2) / (m2 + tpos - cg2)
    js = jac1 + jac2

    last_vals = jnp.concatenate([lax.slice(js, (1023, 0), (1024, 128)),
                                 lax.slice(js, (2047, 0), (2048, 128))],
                                axis=0)
    rl = jnp.roll(last_vals, 1, axis=1)
    lv_h0_last = lax.slice(last_vals, (0, 127), (1, 128))
    prev_last = jnp.where(lane2 == 0,
                          jnp.where(half2 == 0, 0.0, lv_h0_last), rl)
    prev = jnp.where(r10 == 0,
                     jnp.repeat(prev_last.reshape(2, 1, 128), 1024,
                                axis=1).reshape(2048, 128),
                     jnp.roll(js, 1, axis=0))

    f = jnp.where(e > 0, e + 1.0, jnp.exp(jnp.minimum(e, 0.0)))
    contrib = jnp.where(valid, f * (js - prev), 0.0)
    out_ref[0, 0] = (0.5 * jnp.sum(contrib)).reshape(1, 1)


def kernel(score, target):
    b, c, h, w = score.shape
    npix = h * w
    nq = b * c
    score3 = score.reshape(nq, npix // 128, 128)
    target3 = target.astype(jnp.int32).reshape(b, npix // 128, 128)

    a_sorted = pl.pallas_call(
        _sort_a_body,
        grid=(nq // 2,),
        in_specs=[
            pl.BlockSpec((2, 1152, 128), lambda q: (q, 0, 0)),
            pl.BlockSpec((1, 1152, 128), lambda q: ((2 * q) // 5, 0, 0)),
            pl.BlockSpec((1, 1152, 128), lambda q: ((2 * q + 1) // 5, 0, 0)),
        ],
        out_specs=pl.BlockSpec((1, 2048, 128), lambda q: (q, 0, 0)),
        out_shape=jax.ShapeDtypeStruct((nq // 2, 2048, 128), jnp.int32),
    )(score3, target3, target3)
    a_sorted = a_sorted.reshape(nq, 1024, 128)

    b_sorted = pl.pallas_call(
        _sort_b_body,
        grid=(nq // 8,),
        in_specs=[
            pl.BlockSpec((8, 128, 128), lambda g: (g, 8, 0)),
            pl.BlockSpec((8, 128, 128), lambda g: (0, 8, 0)),
        ],
        out_specs=pl.BlockSpec((1, 8, 1024, 16), lambda g: (g, 0, 0, 0)),
        out_shape=jax.ShapeDtypeStruct((nq // 8, 8, 1024, 16), jnp.int32),
    )(score3, target3)

    out = pl.pallas_call(
        functools.partial(_merge_body, npix=npix),
        grid=(nq,),
        in_specs=[
            pl.BlockSpec((1, 1024, 128), lambda q: (q, 0, 0)),
            pl.BlockSpec((1, 1, 1024, 16), lambda q: (q // 8, q % 8, 0, 0)),
        ],
        out_specs=pl.BlockSpec((1, 1, 1, 1), lambda q: (q % 5, q // 5, 0, 0)),
        out_shape=jax.ShapeDtypeStruct((c, b, 1, 1), jnp.float32),
    )(a_sorted, b_sorted)
    return tuple(out[i].reshape(b, 1) for i in range(c))
